# baseline (speedup 1.0000x reference)
import functools

import jax
import jax.numpy as jnp
from jax.experimental import pallas as pl
from jax.experimental.pallas import tpu as pltpu
from jax.experimental.pallas import tpu_sc as plsc

_K = 171
_IGNORE = 255
_M2 = 4096
_ROWS_PER_CHUNK = 1


def _make_pack_body(nc, ns, rows_per_worker, hw):
    def body(src_ref, dst_ref, vin0, vin1, vout0, vout1, si0, si1, so0, so1):
        c_per_b = src_ref.shape[1]
        wid = jax.lax.axis_index("s") * nc + jax.lax.axis_index("c")
        row0 = wid * rows_per_worker
        bb = row0 // c_per_b
        c0 = row0 % c_per_b
        vins, vouts = (vin0, vin1), (vout0, vout1)
        sis, sos = (si0, si1), (so0, so1)

        def in_copy(k, buf):
            return pltpu.make_async_copy(
                src_ref.at[bb, c0 + k, :], vins[buf], sis[buf])

        def out_copy(k, buf):
            return pltpu.make_async_copy(
                vouts[buf], dst_ref.at[bb, c0 + k, :], sos[buf])

        in_copy(0, 0).start()

        for k in range(rows_per_worker):
            buf = k % 2
            if k + 1 < rows_per_worker:
                in_copy(k + 1, (k + 1) % 2).start()
            in_copy(k, buf).wait()
            if k >= 2:
                out_copy(k - 2, buf).wait()

            vin, vout = vins[buf], vouts[buf]

            def inner(j, carry):
                a = vin[pl.ds(j * 32, 16)]
                b = vin[pl.ds(j * 32 + 16, 16)]
                packed = plsc.pack(a, b, format=plsc.PackFormat.INTERLEAVED)
                vout[pl.ds(j * 16, 16)] = plsc.bitcast(packed, jnp.float32)
                return carry

            jax.lax.fori_loop(0, hw // 32, inner, 0, unroll=8)
            out_copy(k, buf).start()

        out_copy(rows_per_worker - 2, rows_per_worker % 2).wait()
        out_copy(rows_per_worker - 1, (rows_per_worker - 1) % 2).wait()

    return body


def _pack_pairs(feats):
    b, c, hw = feats.shape
    try:
        mesh = plsc.VectorSubcoreMesh(core_axis_name="c", subcore_axis_name="s")
        nc, ns = mesh.num_cores, mesh.num_subcores
    except Exception:
        mesh = plsc.VectorSubcoreMesh(core_axis_name="c", subcore_axis_name="s",
                                      num_cores=2, num_subcores=16)
        nc, ns = 2, 16
    rows_per_worker = (b * c) // (nc * ns)

    fn = pl.kernel(
        _make_pack_body(nc, ns, rows_per_worker, hw),
        out_type=jax.ShapeDtypeStruct((b, c, hw // 2), jnp.float32),
        mesh=mesh,
        compiler_params=pltpu.CompilerParams(needs_layout_passes=False),
        scratch_types=[
            pltpu.VMEM((hw,), jnp.float32),
            pltpu.VMEM((hw,), jnp.float32),
            pltpu.VMEM((hw // 2,), jnp.float32),
            pltpu.VMEM((hw // 2,), jnp.float32),
            pltpu.SemaphoreType.DMA,
            pltpu.SemaphoreType.DMA,
            pltpu.SemaphoreType.DMA,
            pltpu.SemaphoreType.DMA,
        ],
    )
    return fn(feats)


def _decode(words):
    u = jax.lax.bitcast_convert_type(words, jnp.uint32)
    lo = jax.lax.bitcast_convert_type(u << 16, jnp.float32)
    hi = jax.lax.bitcast_convert_type(u & jnp.uint32(0xFFFF0000), jnp.float32)
    return lo, hi


def _stream_terms(f, lab, p, inv_t):
    nrm2 = jnp.sum(f * f, axis=0, keepdims=True)
    inv_norm = jax.lax.rsqrt(jnp.maximum(nrm2, 1e-24))
    s = jax.lax.dot_general(
        p, f, (((1,), (0,)), ((), ())),
        preferred_element_type=jnp.float32,
    )
    logits = s * inv_norm * inv_t.T
    mx = jnp.max(logits, axis=0, keepdims=True)
    lse = jnp.log(jnp.sum(jnp.exp(logits - mx), axis=0, keepdims=True)) + mx
    safe_lab = jnp.clip(lab, 0, _K - 1)
    kiota = jax.lax.broadcasted_iota(jnp.int32, logits.shape, 0)
    picked = jnp.sum(jnp.where(kiota == safe_lab, logits, 0.0),
                     axis=0, keepdims=True)
    valid = (lab != _IGNORE).astype(jnp.float32)
    return jnp.sum((lse - picked) * valid), jnp.sum(valid)


def _loss_kernel(w_ref, la_ref, lb_ref, p_ref, t_ref, loss_ref, acc_ref):
    i = pl.program_id(0)
    n_steps = pl.num_programs(0)

    @pl.when(i == 0)
    def _init():
        acc_ref[0] = 0.0
        acc_ref[1] = 0.0

    lab_a = la_ref[0]
    lab_b = lb_ref[0]
    p = p_ref[...]
    t = t_ref[...]
    inv_t = 1.0 / jnp.clip(t, 0.01, 1.0)

    fa, fb = _decode(w_ref[0])
    la, wa = _stream_terms(fa, lab_a, p, inv_t)
    lb, wb = _stream_terms(fb, lab_b, p, inv_t)

    acc_ref[0] += la + lb
    acc_ref[1] += wa + wb

    @pl.when(i == n_steps - 1)
    def _final():
        loss_ref[...] = jnp.broadcast_to(
            acc_ref[0] / jnp.maximum(acc_ref[1], 1.0), (1, 1))


@functools.partial(jax.jit, static_argnames=())
def kernel(features, labels, prototypes, class_temperature):
    b, c, h, w = features.shape
    k = prototypes.shape[0]
    hw = h * w
    hw2 = hw // 2
    nm = hw2 // _M2
    grid = b * nm

    words = _pack_pairs(features.reshape(b, c, hw))
    lab2 = labels.reshape(-1, 32)
    lab_a = lab2[:, :16].reshape(grid, 1, _M2)
    lab_b = lab2[:, 16:].reshape(grid, 1, _M2)
    p_bf = prototypes.astype(jnp.bfloat16).astype(jnp.float32)
    temps = class_temperature.reshape(1, k)

    loss = pl.pallas_call(
        _loss_kernel,
        grid=(grid,),
        in_specs=[
            pl.BlockSpec((1, c, _M2), lambda i: (i // nm, 0, i % nm)),
            pl.BlockSpec((1, 1, _M2), lambda i: (i, 0, 0)),
            pl.BlockSpec((1, 1, _M2), lambda i: (i, 0, 0)),
            pl.BlockSpec((k, c), lambda i: (0, 0)),
            pl.BlockSpec((1, k), lambda i: (0, 0)),
        ],
        out_specs=pl.BlockSpec((1, 1), lambda i: (0, 0)),
        out_shape=jax.ShapeDtypeStruct((1, 1), jnp.float32),
        scratch_shapes=[pltpu.SMEM((2,), jnp.float32)],
        compiler_params=pltpu.CompilerParams(
            dimension_semantics=("arbitrary",),
        ),
    )(words, lab_a, lab_b, p_bf, temps)

    return loss[0, 0]


# speedup vs baseline: 1.0184x; 1.0184x over previous
import functools

import jax
import jax.numpy as jnp
from jax.experimental import pallas as pl
from jax.experimental.pallas import tpu as pltpu
from jax.experimental.pallas import tpu_sc as plsc

_K = 171
_IGNORE = 255
_M2 = 4096
_ROWS_PER_CHUNK = 1


def _make_pack_body(nc, ns, rows_per_worker, hw):
    def body(src_ref, dst_ref, vin0, vin1, vout0, vout1, si0, si1, so0, so1):
        c_per_b = src_ref.shape[1]
        wid = jax.lax.axis_index("s") * nc + jax.lax.axis_index("c")
        row0 = wid * rows_per_worker
        bb = row0 // c_per_b
        c0 = row0 % c_per_b
        vins, vouts = (vin0, vin1), (vout0, vout1)
        sis, sos = (si0, si1), (so0, so1)

        def in_copy(k, buf):
            return pltpu.make_async_copy(
                src_ref.at[bb, c0 + k, :], vins[buf], sis[buf])

        def out_copy(k, buf):
            return pltpu.make_async_copy(
                vouts[buf], dst_ref.at[bb, c0 + k, :], sos[buf])

        in_copy(0, 0).start()

        for k in range(rows_per_worker):
            buf = k % 2
            if k + 1 < rows_per_worker:
                in_copy(k + 1, (k + 1) % 2).start()
            in_copy(k, buf).wait()
            if k >= 2:
                out_copy(k - 2, buf).wait()

            vin, vout = vins[buf], vouts[buf]

            def inner(j, carry):
                a = vin[pl.ds(j * 32, 16)]
                b = vin[pl.ds(j * 32 + 16, 16)]
                packed = plsc.pack(a, b, format=plsc.PackFormat.INTERLEAVED)
                vout[pl.ds(j * 16, 16)] = plsc.bitcast(packed, jnp.float32)
                return carry

            jax.lax.fori_loop(0, hw // 32, inner, 0, unroll=16)
            out_copy(k, buf).start()

        out_copy(rows_per_worker - 2, rows_per_worker % 2).wait()
        out_copy(rows_per_worker - 1, (rows_per_worker - 1) % 2).wait()

    return body


def _pack_pairs(feats):
    b, c, hw = feats.shape
    try:
        mesh = plsc.VectorSubcoreMesh(core_axis_name="c", subcore_axis_name="s")
        nc, ns = mesh.num_cores, mesh.num_subcores
    except Exception:
        mesh = plsc.VectorSubcoreMesh(core_axis_name="c", subcore_axis_name="s",
                                      num_cores=2, num_subcores=16)
        nc, ns = 2, 16
    rows_per_worker = (b * c) // (nc * ns)

    fn = pl.kernel(
        _make_pack_body(nc, ns, rows_per_worker, hw),
        out_type=jax.ShapeDtypeStruct((b, c, hw // 2), jnp.float32),
        mesh=mesh,
        compiler_params=pltpu.CompilerParams(needs_layout_passes=False),
        scratch_types=[
            pltpu.VMEM((hw,), jnp.float32),
            pltpu.VMEM((hw,), jnp.float32),
            pltpu.VMEM((hw // 2,), jnp.float32),
            pltpu.VMEM((hw // 2,), jnp.float32),
            pltpu.SemaphoreType.DMA,
            pltpu.SemaphoreType.DMA,
            pltpu.SemaphoreType.DMA,
            pltpu.SemaphoreType.DMA,
        ],
    )
    return fn(feats)


def _decode(words):
    u = jax.lax.bitcast_convert_type(words, jnp.uint32)
    lo = jax.lax.bitcast_convert_type(u << 16, jnp.float32)
    hi = jax.lax.bitcast_convert_type(u & jnp.uint32(0xFFFF0000), jnp.float32)
    return lo, hi


def _stream_terms(f, lab, p, inv_t):
    nrm2 = jnp.sum(f * f, axis=0, keepdims=True)
    inv_norm = jax.lax.rsqrt(jnp.maximum(nrm2, 1e-24))
    s = jax.lax.dot_general(
        p, f, (((1,), (0,)), ((), ())),
        preferred_element_type=jnp.float32,
    )
    logits = s * inv_norm * inv_t.T
    mx = jnp.max(logits, axis=0, keepdims=True)
    lse = jnp.log(jnp.sum(jnp.exp(logits - mx), axis=0, keepdims=True)) + mx
    safe_lab = jnp.clip(lab, 0, _K - 1)
    kiota = jax.lax.broadcasted_iota(jnp.int32, logits.shape, 0)
    picked = jnp.sum(jnp.where(kiota == safe_lab, logits, 0.0),
                     axis=0, keepdims=True)
    valid = (lab != _IGNORE).astype(jnp.float32)
    return jnp.sum((lse - picked) * valid), jnp.sum(valid)


def _loss_kernel(w_ref, la_ref, lb_ref, p_ref, t_ref, ls_ref, ws_ref, acc_ref):
    i = pl.program_id(0)
    n_steps = pl.num_programs(0)

    @pl.when(i == 0)
    def _init():
        acc_ref[0] = 0.0
        acc_ref[1] = 0.0

    lab_a = la_ref[0]
    lab_b = lb_ref[0]
    p = p_ref[...]
    t = t_ref[...]
    inv_t = 1.0 / jnp.clip(t, 0.01, 1.0)

    fa, fb = _decode(w_ref[0])
    la, wa = _stream_terms(fa, lab_a, p, inv_t)
    lb, wb = _stream_terms(fb, lab_b, p, inv_t)

    acc_ref[0] += la + lb
    acc_ref[1] += wa + wb

    @pl.when(i == n_steps - 1)
    def _final():
        ls_ref[...] = jnp.broadcast_to(acc_ref[0], (1, 1))
        ws_ref[...] = jnp.broadcast_to(acc_ref[1], (1, 1))


def _piece_sums(words, lab_a, lab_b, p_bf, temps):
    _, c, hw2 = words.shape
    k = p_bf.shape[0]
    nm = hw2 // _M2

    sums = pl.pallas_call(
        _loss_kernel,
        grid=(nm,),
        in_specs=[
            pl.BlockSpec((1, c, _M2), lambda i: (0, 0, i)),
            pl.BlockSpec((1, 1, _M2), lambda i: (i, 0, 0)),
            pl.BlockSpec((1, 1, _M2), lambda i: (i, 0, 0)),
            pl.BlockSpec((k, c), lambda i: (0, 0)),
            pl.BlockSpec((1, k), lambda i: (0, 0)),
        ],
        out_specs=[pl.BlockSpec((1, 1), lambda i: (0, 0)),
                   pl.BlockSpec((1, 1), lambda i: (0, 0))],
        out_shape=[jax.ShapeDtypeStruct((1, 1), jnp.float32),
                   jax.ShapeDtypeStruct((1, 1), jnp.float32)],
        scratch_shapes=[pltpu.SMEM((2,), jnp.float32)],
        compiler_params=pltpu.CompilerParams(
            dimension_semantics=("arbitrary",),
        ),
    )(words, lab_a, lab_b, p_bf, temps)
    return sums


@functools.partial(jax.jit, static_argnames=())
def kernel(features, labels, prototypes, class_temperature):
    b, c, h, w = features.shape
    k = prototypes.shape[0]
    hw = h * w
    hw2 = hw // 2
    nm = hw2 // _M2

    p_bf = prototypes.astype(jnp.bfloat16).astype(jnp.float32)
    temps = class_temperature.reshape(1, k)
    lab3 = labels.reshape(b, -1, 32)

    loss_sum = jnp.float32(0.0)
    w_sum = jnp.float32(0.0)
    for bi in range(b):
        words = _pack_pairs(features[bi].reshape(1, c, hw))
        lab_a = lab3[bi, :, :16].reshape(nm, 1, _M2)
        lab_b = lab3[bi, :, 16:].reshape(nm, 1, _M2)
        ls, ws = _piece_sums(words, lab_a, lab_b, p_bf, temps)
        loss_sum = loss_sum + ls[0, 0]
        w_sum = w_sum + ws[0, 0]

    return loss_sum / jnp.maximum(w_sum, 1.0)


# speedup vs baseline: 1.0229x; 1.0044x over previous
import functools

import jax
import jax.numpy as jnp
from jax.experimental import pallas as pl
from jax.experimental.pallas import tpu as pltpu
from jax.experimental.pallas import tpu_sc as plsc

_K = 171
_IGNORE = 255
_M2 = 4096
_ROWS_PER_CHUNK = 1


def _make_pack_body(nc, ns, rows_per_worker, hw):
    def body(src_ref, dst_ref, vin0, vin1, vout0, vout1, si0, si1, so0, so1):
        c_per_b = src_ref.shape[1]
        wid = jax.lax.axis_index("s") * nc + jax.lax.axis_index("c")
        row0 = wid * rows_per_worker
        bb = row0 // c_per_b
        c0 = row0 % c_per_b
        vins, vouts = (vin0, vin1), (vout0, vout1)
        sis, sos = (si0, si1), (so0, so1)

        def in_copy(k, buf):
            return pltpu.make_async_copy(
                src_ref.at[bb, c0 + k, :], vins[buf], sis[buf])

        def out_copy(k, buf):
            return pltpu.make_async_copy(
                vouts[buf], dst_ref.at[bb, c0 + k, :], sos[buf])

        in_copy(0, 0).start()

        for k in range(rows_per_worker):
            buf = k % 2
            if k + 1 < rows_per_worker:
                in_copy(k + 1, (k + 1) % 2).start()
            in_copy(k, buf).wait()
            if k >= 2:
                out_copy(k - 2, buf).wait()

            vin, vout = vins[buf], vouts[buf]

            def inner(j, carry):
                a = vin[pl.ds(j * 32, 16)]
                b = vin[pl.ds(j * 32 + 16, 16)]
                packed = plsc.pack(a, b, format=plsc.PackFormat.INTERLEAVED)
                vout[pl.ds(j * 16, 16)] = plsc.bitcast(packed, jnp.float32)
                return carry

            jax.lax.fori_loop(0, hw // 32, inner, 0, unroll=16)
            out_copy(k, buf).start()

        out_copy(rows_per_worker - 2, rows_per_worker % 2).wait()
        out_copy(rows_per_worker - 1, (rows_per_worker - 1) % 2).wait()

    return body


def _pack_pairs(feats):
    b, c, hw = feats.shape
    try:
        mesh = plsc.VectorSubcoreMesh(core_axis_name="c", subcore_axis_name="s")
        nc, ns = mesh.num_cores, mesh.num_subcores
    except Exception:
        mesh = plsc.VectorSubcoreMesh(core_axis_name="c", subcore_axis_name="s",
                                      num_cores=2, num_subcores=16)
        nc, ns = 2, 16
    rows_per_worker = (b * c) // (nc * ns)

    fn = pl.kernel(
        _make_pack_body(nc, ns, rows_per_worker, hw),
        out_type=jax.ShapeDtypeStruct((b, c, hw // 2), jnp.float32),
        mesh=mesh,
        compiler_params=pltpu.CompilerParams(needs_layout_passes=False),
        scratch_types=[
            pltpu.VMEM((hw,), jnp.float32),
            pltpu.VMEM((hw,), jnp.float32),
            pltpu.VMEM((hw // 2,), jnp.float32),
            pltpu.VMEM((hw // 2,), jnp.float32),
            pltpu.SemaphoreType.DMA,
            pltpu.SemaphoreType.DMA,
            pltpu.SemaphoreType.DMA,
            pltpu.SemaphoreType.DMA,
        ],
    )
    return fn(feats)


def _decode(words):
    u = jax.lax.bitcast_convert_type(words, jnp.uint32)
    lo = jax.lax.bitcast_convert_type(u << 16, jnp.float32)
    hi = jax.lax.bitcast_convert_type(u & jnp.uint32(0xFFFF0000), jnp.float32)
    return lo, hi


def _stream_terms(f, lab, p, inv_t):
    nrm2 = jnp.sum(f * f, axis=0, keepdims=True)
    inv_norm = jax.lax.rsqrt(jnp.maximum(nrm2, 1e-24))
    s = jax.lax.dot_general(
        p, f, (((1,), (0,)), ((), ())),
        preferred_element_type=jnp.float32,
    )
    logits = s * inv_norm * inv_t.T
    mx = jnp.max(logits, axis=0, keepdims=True)
    lse = jnp.log(jnp.sum(jnp.exp(logits - mx), axis=0, keepdims=True)) + mx
    safe_lab = jnp.clip(lab, 0, _K - 1)
    kiota = jax.lax.broadcasted_iota(jnp.int32, logits.shape, 0)
    picked = jnp.sum(jnp.where(kiota == safe_lab, logits, 0.0),
                     axis=0, keepdims=True)
    valid = (lab != _IGNORE).astype(jnp.float32)
    return jnp.sum((lse - picked) * valid), jnp.sum(valid)


def _loss_kernel(w_ref, la_ref, lb_ref, p_ref, t_ref, ls_ref, ws_ref, acc_ref):
    i = pl.program_id(0)
    n_steps = pl.num_programs(0)

    @pl.when(i == 0)
    def _init():
        acc_ref[0] = 0.0
        acc_ref[1] = 0.0

    lab_a = la_ref[0]
    lab_b = lb_ref[0]
    p = p_ref[...]
    t = t_ref[...]
    inv_t = 1.0 / jnp.clip(t, 0.01, 1.0)

    fa, fb = _decode(w_ref[0])
    la, wa = _stream_terms(fa, lab_a, p, inv_t)
    lb, wb = _stream_terms(fb, lab_b, p, inv_t)

    acc_ref[0] += la + lb
    acc_ref[1] += wa + wb

    @pl.when(i == n_steps - 1)
    def _final():
        ls_ref[...] = jnp.broadcast_to(acc_ref[0], (1, 1))
        ws_ref[...] = jnp.broadcast_to(acc_ref[1], (1, 1))


def _piece_sums(words, lab_a, lab_b, p_bf, temps):
    _, c, hw2 = words.shape
    k = p_bf.shape[0]
    nm = hw2 // _M2

    sums = pl.pallas_call(
        _loss_kernel,
        grid=(nm,),
        in_specs=[
            pl.BlockSpec((1, c, _M2), lambda i: (0, 0, i)),
            pl.BlockSpec((1, 1, _M2), lambda i: (i, 0, 0)),
            pl.BlockSpec((1, 1, _M2), lambda i: (i, 0, 0)),
            pl.BlockSpec((k, c), lambda i: (0, 0)),
            pl.BlockSpec((1, k), lambda i: (0, 0)),
        ],
        out_specs=[pl.BlockSpec((1, 1), lambda i: (0, 0)),
                   pl.BlockSpec((1, 1), lambda i: (0, 0))],
        out_shape=[jax.ShapeDtypeStruct((1, 1), jnp.float32),
                   jax.ShapeDtypeStruct((1, 1), jnp.float32)],
        scratch_shapes=[pltpu.SMEM((2,), jnp.float32)],
        compiler_params=pltpu.CompilerParams(
            dimension_semantics=("arbitrary",),
        ),
    )(words, lab_a, lab_b, p_bf, temps)
    return sums


@functools.partial(jax.jit, static_argnames=())
def kernel(features, labels, prototypes, class_temperature):
    b, c, h, w = features.shape
    k = prototypes.shape[0]
    hw = h * w
    hw2 = hw // 2
    nm = hw2 // _M2

    p_bf = prototypes.astype(jnp.bfloat16).astype(jnp.float32)
    temps = class_temperature.reshape(1, k)
    lab3 = labels.reshape(b, -1, 32)

    words_list = [_pack_pairs(features[bi].reshape(1, c, hw))
                  for bi in range(b)]

    loss_sum = jnp.float32(0.0)
    w_sum = jnp.float32(0.0)
    for bi in range(b):
        lab_a = lab3[bi, :, :16].reshape(nm, 1, _M2)
        lab_b = lab3[bi, :, 16:].reshape(nm, 1, _M2)
        ls, ws = _piece_sums(words_list[bi], lab_a, lab_b, p_bf, temps)
        loss_sum = loss_sum + ls[0, 0]
        w_sum = w_sum + ws[0, 0]

    return loss_sum / jnp.maximum(w_sum, 1.0)


# speedup vs baseline: 1.5282x; 1.4940x over previous
import functools

import jax
import jax.numpy as jnp
from jax.experimental import pallas as pl
from jax.experimental.pallas import tpu as pltpu

_K = 171
_IGNORE = 255
_M = 8192


def _loss_kernel(f_ref, lab_ref, p_ref, t_ref, loss_ref, acc_ref):
    i = pl.program_id(0)
    n_steps = pl.num_programs(0)

    @pl.when(i == 0)
    def _init():
        acc_ref[0] = 0.0
        acc_ref[1] = 0.0

    f = f_ref[0]
    lab = lab_ref[0]
    p = p_ref[...]
    t = t_ref[...]

    nrm2 = jnp.sum(f * f, axis=0, keepdims=True)
    inv_norm = jax.lax.rsqrt(jnp.maximum(nrm2, 1e-24))

    s = jax.lax.dot_general(
        p, f, (((1,), (0,)), ((), ())),
        preferred_element_type=jnp.float32,
        precision=jax.lax.Precision.HIGHEST,
    )

    inv_t = 1.0 / jnp.clip(t, 0.01, 1.0)
    logits = s * inv_norm * inv_t.T

    mx = jnp.max(logits, axis=0, keepdims=True)
    lse = jnp.log(jnp.sum(jnp.exp(logits - mx), axis=0, keepdims=True)) + mx

    safe_lab = jnp.clip(lab, 0, _K - 1)
    kiota = jax.lax.broadcasted_iota(jnp.int32, logits.shape, 0)
    picked = jnp.sum(jnp.where(kiota == safe_lab, logits, 0.0),
                     axis=0, keepdims=True)

    valid = (lab != _IGNORE).astype(jnp.float32)
    acc_ref[0] += jnp.sum((lse - picked) * valid)
    acc_ref[1] += jnp.sum(valid)

    @pl.when(i == n_steps - 1)
    def _final():
        loss_ref[...] = jnp.broadcast_to(
            acc_ref[0] / jnp.maximum(acc_ref[1], 1.0), (1, 1))


@functools.partial(jax.jit, static_argnames=())
def kernel(features, labels, prototypes, class_temperature):
    b, c, h, w = features.shape
    k = prototypes.shape[0]
    hw = h * w
    nm = hw // _M
    grid = b * nm

    feats = features.reshape(b, c, hw)
    labs = labels.reshape(grid, 1, _M)
    temps = class_temperature.reshape(1, k)

    loss = pl.pallas_call(
        _loss_kernel,
        grid=(grid,),
        in_specs=[
            pl.BlockSpec((1, c, _M), lambda i: (i // nm, 0, i % nm)),
            pl.BlockSpec((1, 1, _M), lambda i: (i, 0, 0)),
            pl.BlockSpec((k, c), lambda i: (0, 0)),
            pl.BlockSpec((1, k), lambda i: (0, 0)),
        ],
        out_specs=pl.BlockSpec((1, 1), lambda i: (0, 0)),
        out_shape=jax.ShapeDtypeStruct((1, 1), jnp.float32),
        scratch_shapes=[pltpu.SMEM((2,), jnp.float32)],
        compiler_params=pltpu.CompilerParams(
            dimension_semantics=("arbitrary",),
        ),
    )(feats, labs, prototypes, temps)

    return loss[0, 0]


# speedup vs baseline: 1.9356x; 1.2666x over previous
import functools

import jax
import jax.numpy as jnp
from jax.experimental import pallas as pl
from jax.experimental.pallas import tpu as pltpu

_K = 171
_IGNORE = 255
_M = 8192


def _loss_kernel(f_ref, lab_ref, p_ref, t_ref, loss_ref, acc_ref):
    i = pl.program_id(0)
    n_steps = pl.num_programs(0)

    @pl.when(i == 0)
    def _init():
        acc_ref[0] = 0.0
        acc_ref[1] = 0.0

    f = f_ref[0]
    lab = lab_ref[0]
    p = p_ref[...]
    t = t_ref[...]

    nrm2 = jnp.sum(f * f, axis=0, keepdims=True)
    inv_norm = jax.lax.rsqrt(jnp.maximum(nrm2, 1e-24))

    s = jax.lax.dot_general(
        p, f, (((1,), (0,)), ((), ())),
        preferred_element_type=jnp.float32,
        precision=jax.lax.Precision.DEFAULT,
    )

    inv_t = 1.0 / jnp.clip(t, 0.01, 1.0)
    logits = s * inv_norm * inv_t.T

    mx = jnp.max(logits, axis=0, keepdims=True)
    lse = jnp.log(jnp.sum(jnp.exp(logits - mx), axis=0, keepdims=True)) + mx

    safe_lab = jnp.clip(lab, 0, _K - 1)
    kiota = jax.lax.broadcasted_iota(jnp.int32, logits.shape, 0)
    picked = jnp.sum(jnp.where(kiota == safe_lab, logits, 0.0),
                     axis=0, keepdims=True)

    valid = (lab != _IGNORE).astype(jnp.float32)
    acc_ref[0] += jnp.sum((lse - picked) * valid)
    acc_ref[1] += jnp.sum(valid)

    @pl.when(i == n_steps - 1)
    def _final():
        loss_ref[...] = jnp.broadcast_to(
            acc_ref[0] / jnp.maximum(acc_ref[1], 1.0), (1, 1))


@functools.partial(jax.jit, static_argnames=())
def kernel(features, labels, prototypes, class_temperature):
    b, c, h, w = features.shape
    k = prototypes.shape[0]
    hw = h * w
    nm = hw // _M
    grid = b * nm

    feats = features.reshape(b, c, hw)
    labs = labels.reshape(grid, 1, _M)
    temps = class_temperature.reshape(1, k)

    loss = pl.pallas_call(
        _loss_kernel,
        grid=(grid,),
        in_specs=[
            pl.BlockSpec((1, c, _M), lambda i: (i // nm, 0, i % nm)),
            pl.BlockSpec((1, 1, _M), lambda i: (i, 0, 0)),
            pl.BlockSpec((k, c), lambda i: (0, 0)),
            pl.BlockSpec((1, k), lambda i: (0, 0)),
        ],
        out_specs=pl.BlockSpec((1, 1), lambda i: (0, 0)),
        out_shape=jax.ShapeDtypeStruct((1, 1), jnp.float32),
        scratch_shapes=[pltpu.SMEM((2,), jnp.float32)],
        compiler_params=pltpu.CompilerParams(
            dimension_semantics=("arbitrary",),
        ),
    )(feats, labs, prototypes, temps)

    return loss[0, 0]


# speedup vs baseline: 1.9367x; 1.0006x over previous
import functools

import jax
import jax.numpy as jnp
from jax.experimental import pallas as pl
from jax.experimental.pallas import tpu as pltpu

_K = 171
_IGNORE = 255
_M = 8192


def _loss_kernel(f_ref, lab_ref, p_ref, t_ref, loss_ref, acc_ref):
    i = pl.program_id(0)
    n_steps = pl.num_programs(0)

    @pl.when(i == 0)
    def _init():
        acc_ref[0] = 0.0
        acc_ref[1] = 0.0

    f = f_ref[0]
    lab = lab_ref[0]
    p = p_ref[...]
    t = t_ref[...]

    nrm2 = jnp.sum(f * f, axis=0, keepdims=True)
    inv_norm = jax.lax.rsqrt(jnp.maximum(nrm2, 1e-24))

    s = jax.lax.dot_general(
        p.astype(jnp.bfloat16), f.astype(jnp.bfloat16),
        (((1,), (0,)), ((), ())),
        preferred_element_type=jnp.float32,
    )

    inv_t = 1.0 / jnp.clip(t, 0.01, 1.0)
    logits = s * inv_norm * inv_t.T

    mx = jnp.max(logits, axis=0, keepdims=True)
    lse = jnp.log(jnp.sum(jnp.exp(logits - mx), axis=0, keepdims=True)) + mx

    safe_lab = jnp.clip(lab, 0, _K - 1)
    kiota = jax.lax.broadcasted_iota(jnp.int32, logits.shape, 0)
    picked = jnp.sum(jnp.where(kiota == safe_lab, logits, 0.0),
                     axis=0, keepdims=True)

    valid = (lab != _IGNORE).astype(jnp.float32)
    acc_ref[0] += jnp.sum((lse - picked) * valid)
    acc_ref[1] += jnp.sum(valid)

    @pl.when(i == n_steps - 1)
    def _final():
        loss_ref[...] = jnp.broadcast_to(
            acc_ref[0] / jnp.maximum(acc_ref[1], 1.0), (1, 1))


@functools.partial(jax.jit, static_argnames=())
def kernel(features, labels, prototypes, class_temperature):
    b, c, h, w = features.shape
    k = prototypes.shape[0]
    hw = h * w
    nm = hw // _M
    grid = b * nm

    feats = features.reshape(b, c, hw)
    labs = labels.reshape(grid, 1, _M)
    temps = class_temperature.reshape(1, k)

    loss = pl.pallas_call(
        _loss_kernel,
        grid=(grid,),
        in_specs=[
            pl.BlockSpec((1, c, _M), lambda i: (i // nm, 0, i % nm)),
            pl.BlockSpec((1, 1, _M), lambda i: (i, 0, 0)),
            pl.BlockSpec((k, c), lambda i: (0, 0)),
            pl.BlockSpec((1, k), lambda i: (0, 0)),
        ],
        out_specs=pl.BlockSpec((1, 1), lambda i: (0, 0)),
        out_shape=jax.ShapeDtypeStruct((1, 1), jnp.float32),
        scratch_shapes=[pltpu.SMEM((2,), jnp.float32)],
        compiler_params=pltpu.CompilerParams(
            dimension_semantics=("arbitrary",),
        ),
    )(feats, labs, prototypes, temps)

    return loss[0, 0]
